# baseline (speedup 1.0000x reference)
import jax
import jax.numpy as jnp
from jax.experimental import pallas as pl
from jax.experimental.pallas import tpu as pltpu

_EPS = 1e-5


def _gct_body(x_ref, a_ref, g_ref, b_ref, o_ref):
    x = x_ref[...]
    sumsq = jnp.sum(x * x, axis=2)
    embed = jnp.sqrt(sumsq + _EPS) * a_ref[...]
    inv = jax.lax.rsqrt(
        jnp.mean(embed * embed, axis=1, keepdims=True) + _EPS
    )
    z = embed * g_ref[...] * inv + b_ref[...]
    gate = 1.0 + jnp.tanh(z)
    o_ref[...] = x * gate[:, :, None]


def kernel(x, alpha, gamma, beta):
    N, C, H, W = x.shape
    HW = H * W
    xr = x.reshape(N, C, HW)
    a2 = alpha.reshape(1, C)
    g2 = gamma.reshape(1, C)
    b2 = beta.reshape(1, C)
    out = pl.pallas_call(
        _gct_body,
        grid=(N,),
        in_specs=[
            pl.BlockSpec((1, C, HW), lambda n: (n, 0, 0)),
            pl.BlockSpec((1, C), lambda n: (0, 0)),
            pl.BlockSpec((1, C), lambda n: (0, 0)),
            pl.BlockSpec((1, C), lambda n: (0, 0)),
        ],
        out_specs=pl.BlockSpec((1, C, HW), lambda n: (n, 0, 0)),
        out_shape=jax.ShapeDtypeStruct((N, C, HW), x.dtype),
        compiler_params=pltpu.CompilerParams(
            dimension_semantics=("parallel",)
        ),
    )(xr, a2, g2, b2)
    return out.reshape(N, C, H, W)
